# baseline (speedup 1.0000x reference)
import functools

import jax
import jax.numpy as jnp
from jax import lax
from jax.experimental import pallas as pl
from jax.experimental.pallas import tpu as pltpu
from jax.experimental.pallas import tpu_sc as plsc

N = 10000
E = 320000
D = 128
NP = 10240
NPT = NP // 16
NW = 32
CH = 64
CHT = 320
CF = 240
CS = CHT - CF
EP = CHT * 16 * CH
KA = 16
KB = 4
NBUF = 4
BM = 1024
BN = 1000
EPS = 1e-16


def _mm1_body(x_ref, w_ref, av_ref, bv_ref, h_ref, as_ref, ad_ref):
    h = jnp.dot(x_ref[...], w_ref[...], preferred_element_type=jnp.float32)
    h_ref[0] = h
    h_ref[1] = h
    as_ref[...] = jnp.dot(h, av_ref[...])
    ad_ref[...] = jnp.dot(h, bv_ref[...])


def _mm2_body(raw_ref, den_ref, b_ref, w_ref, av_ref, bv_ref,
              h_ref, as_ref, ad_ref):
    raw = raw_ref[0] + raw_ref[1]
    den = den_ref[0] + den_ref[1]
    xin = jnp.maximum(raw / (den + EPS) + b_ref[...], 0.0)
    h = jnp.dot(xin, w_ref[...], preferred_element_type=jnp.float32)
    h_ref[0] = h
    h_ref[1] = h
    as_ref[...] = jnp.dot(h, av_ref[...])
    ad_ref[...] = jnp.dot(h, bv_ref[...])


def _final_body(raw_ref, den_ref, b_ref, out_ref):
    raw = raw_ref[0] + raw_ref[1]
    den = den_ref[0] + den_ref[1]
    out_ref[...] = raw / (den + EPS) + b_ref[...]


def _dense1(xp, W, av, bv):
    return pl.pallas_call(
        _mm1_body,
        grid=(NP // BM,),
        in_specs=[
            pl.BlockSpec((BM, D), lambda i: (i, 0)),
            pl.BlockSpec((D, D), lambda i: (0, 0)),
            pl.BlockSpec((D, 1), lambda i: (0, 0)),
            pl.BlockSpec((D, 1), lambda i: (0, 0)),
        ],
        out_specs=[
            pl.BlockSpec((2, BM, D), lambda i: (0, i, 0)),
            pl.BlockSpec((BM, 1), lambda i: (i, 0)),
            pl.BlockSpec((BM, 1), lambda i: (i, 0)),
        ],
        out_shape=[
            jax.ShapeDtypeStruct((2, NP, D), jnp.float32),
            jax.ShapeDtypeStruct((NP, 1), jnp.float32),
            jax.ShapeDtypeStruct((NP, 1), jnp.float32),
        ],
    )(xp, W, av, bv)


def _dense2(raw, den, b, W, av, bv):
    return pl.pallas_call(
        _mm2_body,
        grid=(NP // BM,),
        in_specs=[
            pl.BlockSpec((2, BM, D), lambda i: (0, i, 0)),
            pl.BlockSpec((2, BM, 1), lambda i: (0, i, 0)),
            pl.BlockSpec((1, D), lambda i: (0, 0)),
            pl.BlockSpec((D, D), lambda i: (0, 0)),
            pl.BlockSpec((D, 1), lambda i: (0, 0)),
            pl.BlockSpec((D, 1), lambda i: (0, 0)),
        ],
        out_specs=[
            pl.BlockSpec((2, BM, D), lambda i: (0, i, 0)),
            pl.BlockSpec((BM, 1), lambda i: (i, 0)),
            pl.BlockSpec((BM, 1), lambda i: (i, 0)),
        ],
        out_shape=[
            jax.ShapeDtypeStruct((2, NP, D), jnp.float32),
            jax.ShapeDtypeStruct((NP, 1), jnp.float32),
            jax.ShapeDtypeStruct((NP, 1), jnp.float32),
        ],
    )(raw, den, b, W, av, bv)


def _finalize(raw, den, b):
    return pl.pallas_call(
        _final_body,
        grid=(N // BN,),
        in_specs=[
            pl.BlockSpec((2, BN, D), lambda i: (0, i, 0)),
            pl.BlockSpec((2, BN, 1), lambda i: (0, i, 0)),
            pl.BlockSpec((1, D), lambda i: (0, 0)),
        ],
        out_specs=pl.BlockSpec((BN, D), lambda i: (i, 0)),
        out_shape=jax.ShapeDtypeStruct((N, D), jnp.float32),
    )(raw, den, b)


@functools.cache
def _sc_mesh():
    return plsc.VectorSubcoreMesh(core_axis_name="c", subcore_axis_name="s",
                                  num_cores=2, num_subcores=16)


@functools.cache
def _make_weight_kernel():
    return functools.partial(
        pl.kernel,
        out_type=[
            jax.ShapeDtypeStruct((16 * CHT, CH), jnp.float32),
            jax.ShapeDtypeStruct((2, NP), jnp.float32),
        ],
        mesh=_sc_mesh(),
        compiler_params=pltpu.CompilerParams(needs_layout_passes=False),
        scratch_types=[
            pltpu.VMEM_SHARED((NP,), jnp.float32),
            pltpu.VMEM((NP,), jnp.float32),
            pltpu.VMEM((NP,), jnp.float32),
            pltpu.VMEM((KA, CH), jnp.int32),
            pltpu.VMEM((KA, CH), jnp.int32),
            pltpu.VMEM((KA, CH), jnp.float32),
            pltpu.VMEM((NPT,), jnp.float32),
        ],
    )(_weight_body)


def _weight_body(asv_hbm, adv_hbm, edges_hbm, w_hbm, den_hbm,
                 den_sh, as_l, ad_l, src_k, dst_k, w_k, zbuf1):
    cid = lax.axis_index("c")
    sid = lax.axis_index("s")
    rowbase = (cid * 16 + sid) * (CHT // 2)
    noa_l = CHT // 2 // KA

    pltpu.sync_copy(asv_hbm, as_l)
    pltpu.sync_copy(adv_hbm, ad_l)

    @pl.loop(0, NPT // 16)
    def _z1(i):
        zbuf1[pl.ds(i * 16, 16)] = jnp.zeros((16,), jnp.float32)

    nslice = pl.ds(sid * NPT, NPT)
    pltpu.sync_copy(zbuf1, den_sh.at[nslice])
    plsc.subcore_barrier()

    @pl.loop(0, noa_l)
    def _batch(o):
        bs = pl.ds((rowbase + o * KA), KA)
        pltpu.sync_copy(edges_hbm.at[0].at[bs], src_k)
        pltpu.sync_copy(edges_hbm.at[1].at[bs], dst_k)
        for jj in range(KA):
            base = (rowbase + o * KA + jj) * CH
            for g in range(CH // 16):
                s_idx = src_k[jj, pl.ds(g * 16, 16)]
                d_idx = dst_k[jj, pl.ds(g * 16, 16)]
                e = (plsc.load_gather(as_l, [s_idx])
                     + plsc.load_gather(ad_l, [d_idx]))
                e = jnp.maximum(e, 0.2 * e)
                w = jnp.exp(e)
                gid = base + g * 16 + lax.iota(jnp.int32, 16)
                w_k[jj, pl.ds(g * 16, 16)] = jnp.where(gid < E, w, 0.0)
        pltpu.sync_copy(w_k, w_hbm.at[bs])
        for jj in range(KA):
            pltpu.sync_copy(w_k.at[jj], den_sh.at[dst_k.at[jj]], add=True)

    plsc.subcore_barrier()
    pltpu.sync_copy(den_sh.at[nslice], zbuf1)
    pltpu.sync_copy(zbuf1, den_hbm.at[cid].at[nslice])


@functools.cache
def _make_accum_kernel():
    return functools.partial(
        pl.kernel,
        out_type=jax.ShapeDtypeStruct((2, NP, D), jnp.float32),
        mesh=_sc_mesh(),
        compiler_params=pltpu.CompilerParams(needs_layout_passes=False),
        scratch_types=[
            pltpu.VMEM_SHARED((NP, D), jnp.float32),
            pltpu.VMEM((KB, CH), jnp.int32),
            pltpu.VMEM((KB, CH), jnp.int32),
            pltpu.VMEM((KB, CH), jnp.int32),
            pltpu.VMEM((KB, CH), jnp.int32),
            pltpu.VMEM((KB, CH), jnp.float32),
            pltpu.VMEM((KB, CH), jnp.float32),
            pltpu.VMEM((CH, D), jnp.float32),
            pltpu.VMEM((CH, D), jnp.float32),
            pltpu.VMEM((CH, D), jnp.float32),
            pltpu.VMEM((CH, D), jnp.float32),
            pltpu.SemaphoreType.DMA,
            pltpu.SemaphoreType.DMA,
            pltpu.SemaphoreType.DMA,
            pltpu.SemaphoreType.DMA,
            pltpu.SemaphoreType.DMA,
            pltpu.SemaphoreType.DMA,
            pltpu.SemaphoreType.DMA,
            pltpu.SemaphoreType.DMA,
        ],
    )(_accum_body)


def _accum_body(h_hbm, w_hbm, edges_hbm, raw_hbm,
                acc, src0, src1, dst0, dst1, w0, w1,
                rows0, rows1, rows2, rows3,
                sg0, sg1, sg2, sg3, ss0, ss1, ss2, ss3):
    cid = lax.axis_index("c")
    sid = lax.axis_index("s")
    rowbase = sid * CHT + cid * CF
    nch_l = jnp.where(cid == 0, CF, CS)
    nob_l = nch_l // KB
    h_sel = h_hbm.at[cid]
    srcs, dsts, ws = (src0, src1), (dst0, dst1), (w0, w1)
    rows = (rows0, rows1, rows2, rows3)
    sg = (sg0, sg1, sg2, sg3)
    ss = (ss0, ss1, ss2, ss3)

    @pl.loop(0, CH)
    def _z(i):
        z = jnp.zeros((16,), jnp.float32)
        for g in range(D // 16):
            rows0[i, pl.ds(g * 16, 16)] = z

    for p in range(NPT // CH):
        pltpu.sync_copy(rows0, acc.at[pl.ds(sid * NPT + p * CH, CH)])
    plsc.subcore_barrier()

    def load_batch(b, s):
        bs = pl.ds((rowbase + b * KB), KB)
        pltpu.sync_copy(edges_hbm.at[0].at[bs], srcs[s])
        pltpu.sync_copy(edges_hbm.at[1].at[bs], dsts[s])
        pltpu.sync_copy(w_hbm.at[bs], ws[s])

    load_batch(0, 0)
    load_batch(1, 1)
    for q in range(NBUF - 1):
        pltpu.async_copy(h_sel.at[srcs[0].at[q]], rows[q], sg[q])

    @pl.loop(0, nob_l // 2)
    def _pair(k):
        for half in range(2):
            b = 2 * k + half
            for jj in range(KB):
                j = b * KB + jj
                @pl.when(j > 0)
                def _w0():
                    pltpu.make_async_copy(
                        rows[(jj + NBUF - 1) % NBUF], acc.at[dsts[half].at[jj]],
                        ss[(jj + NBUF - 1) % NBUF]).wait()
                nrow = (jj + NBUF - 1) % KB
                nset = half if jj == 0 else 1 - half

                @pl.when(j + NBUF - 1 < nch_l)
                def _g1():
                    pltpu.async_copy(
                        h_sel.at[srcs[nset].at[nrow]],
                        rows[(jj + NBUF - 1) % NBUF], sg[(jj + NBUF - 1) % NBUF])
                pltpu.make_async_copy(
                    h_sel.at[srcs[half].at[jj]], rows[jj], sg[jj]).wait()

                @pl.loop(0, CH // 16)
                def _scale(bb):
                    w16 = ws[half][jj, pl.ds(bb * 16, 16)]
                    for i in range(16):
                        s = w16[i]
                        r = bb * 16 + i
                        for gg in range(D // 16):
                            rows[jj][r, pl.ds(gg * 16, 16)] = (
                                rows[jj][r, pl.ds(gg * 16, 16)] * s)

                pltpu.async_copy(rows[jj], acc.at[dsts[half].at[jj]],
                                 ss[jj], add=True)
            @pl.when(b + 2 < nob_l)
            def _r2():
                load_batch(b + 2, half)

    pltpu.make_async_copy(rows[NBUF - 1], acc.at[dsts[1].at[KB - 1]],
                          ss[NBUF - 1]).wait()
    plsc.subcore_barrier()

    for p in range(NPT // CH):
        rs = pl.ds(sid * NPT + p * CH, CH)
        pltpu.sync_copy(acc.at[rs], rows0)
        pltpu.sync_copy(rows0, raw_hbm.at[cid].at[rs])


def kernel(x, edge_index, W1, a_src1, a_dst1, b1, W2, a_src2, a_dst2, b2):
    ei = jnp.pad(edge_index.astype(jnp.int32), ((0, 0), (0, EP - E)))
    ei = ei.reshape(2, 16 * CHT, CH)
    xp = jnp.pad(x, ((0, NP - N), (0, 0)))

    weight_kernel = _make_weight_kernel()
    accum_kernel = _make_accum_kernel()

    h, asv, adv = _dense1(xp, W1, a_src1.reshape(D, 1), a_dst1.reshape(D, 1))
    wv, den = weight_kernel(asv.reshape(NP), adv.reshape(NP), ei)
    raw = accum_kernel(h, wv, ei)

    h2, asv2, adv2 = _dense2(raw, den.reshape(2, NP, 1), b1.reshape(1, D), W2,
                             a_src2.reshape(D, 1), a_dst2.reshape(D, 1))
    wv2, den2 = weight_kernel(asv2.reshape(NP), adv2.reshape(NP), ei)
    raw2 = accum_kernel(h2, wv2, ei)

    return _finalize(raw2[:, :N], den2.reshape(2, NP, 1)[:, :N],
                     b2.reshape(1, D))


# speedup vs baseline: 1.1156x; 1.1156x over previous
import functools

import jax
import jax.numpy as jnp
from jax import lax
from jax.experimental import pallas as pl
from jax.experimental.pallas import tpu as pltpu
from jax.experimental.pallas import tpu_sc as plsc

N = 10000
E = 320000
D = 128
NP = 10240
NPT = NP // 16
NW = 32
CH = 128
CHT = 160
CF = 120
CS = CHT - CF
EP = CHT * 16 * CH
KA = 8
KB = 4
BM = 1024
BN = 1000
EPS = 1e-16


def _mm1_body(x_ref, w_ref, av_ref, bv_ref, h_ref, as_ref, ad_ref):
    h = jnp.dot(x_ref[...], w_ref[...], preferred_element_type=jnp.float32)
    h_ref[0] = h
    h_ref[1] = h
    as_ref[...] = jnp.dot(h, av_ref[...])
    ad_ref[...] = jnp.dot(h, bv_ref[...])


def _mm2_body(raw_ref, den_ref, b_ref, w_ref, av_ref, bv_ref,
              h_ref, as_ref, ad_ref):
    raw = raw_ref[0] + raw_ref[1]
    den = den_ref[0] + den_ref[1]
    xin = jnp.maximum(raw / (den + EPS) + b_ref[...], 0.0)
    h = jnp.dot(xin, w_ref[...], preferred_element_type=jnp.float32)
    h_ref[0] = h
    h_ref[1] = h
    as_ref[...] = jnp.dot(h, av_ref[...])
    ad_ref[...] = jnp.dot(h, bv_ref[...])


def _final_body(raw_ref, den_ref, b_ref, out_ref):
    raw = raw_ref[0] + raw_ref[1]
    den = den_ref[0] + den_ref[1]
    out_ref[...] = raw / (den + EPS) + b_ref[...]


def _dense1(xp, W, av, bv):
    return pl.pallas_call(
        _mm1_body,
        grid=(NP // BM,),
        in_specs=[
            pl.BlockSpec((BM, D), lambda i: (i, 0)),
            pl.BlockSpec((D, D), lambda i: (0, 0)),
            pl.BlockSpec((D, 1), lambda i: (0, 0)),
            pl.BlockSpec((D, 1), lambda i: (0, 0)),
        ],
        out_specs=[
            pl.BlockSpec((2, BM, D), lambda i: (0, i, 0)),
            pl.BlockSpec((BM, 1), lambda i: (i, 0)),
            pl.BlockSpec((BM, 1), lambda i: (i, 0)),
        ],
        out_shape=[
            jax.ShapeDtypeStruct((2, NP, D), jnp.float32),
            jax.ShapeDtypeStruct((NP, 1), jnp.float32),
            jax.ShapeDtypeStruct((NP, 1), jnp.float32),
        ],
    )(xp, W, av, bv)


def _dense2(raw, den, b, W, av, bv):
    return pl.pallas_call(
        _mm2_body,
        grid=(NP // BM,),
        in_specs=[
            pl.BlockSpec((2, BM, D), lambda i: (0, i, 0)),
            pl.BlockSpec((2, BM, 1), lambda i: (0, i, 0)),
            pl.BlockSpec((1, D), lambda i: (0, 0)),
            pl.BlockSpec((D, D), lambda i: (0, 0)),
            pl.BlockSpec((D, 1), lambda i: (0, 0)),
            pl.BlockSpec((D, 1), lambda i: (0, 0)),
        ],
        out_specs=[
            pl.BlockSpec((2, BM, D), lambda i: (0, i, 0)),
            pl.BlockSpec((BM, 1), lambda i: (i, 0)),
            pl.BlockSpec((BM, 1), lambda i: (i, 0)),
        ],
        out_shape=[
            jax.ShapeDtypeStruct((2, NP, D), jnp.float32),
            jax.ShapeDtypeStruct((NP, 1), jnp.float32),
            jax.ShapeDtypeStruct((NP, 1), jnp.float32),
        ],
    )(raw, den, b, W, av, bv)


def _finalize(raw, den, b):
    return pl.pallas_call(
        _final_body,
        grid=(N // BN,),
        in_specs=[
            pl.BlockSpec((2, BN, D), lambda i: (0, i, 0)),
            pl.BlockSpec((2, BN, 1), lambda i: (0, i, 0)),
            pl.BlockSpec((1, D), lambda i: (0, 0)),
        ],
        out_specs=pl.BlockSpec((BN, D), lambda i: (i, 0)),
        out_shape=jax.ShapeDtypeStruct((N, D), jnp.float32),
    )(raw, den, b)


@functools.cache
def _sc_mesh():
    return plsc.VectorSubcoreMesh(core_axis_name="c", subcore_axis_name="s",
                                  num_cores=2, num_subcores=16)


@functools.cache
def _make_weight_kernel():
    return functools.partial(
        pl.kernel,
        out_type=[
            jax.ShapeDtypeStruct((16 * CHT, CH), jnp.float32),
            jax.ShapeDtypeStruct((2, NP), jnp.float32),
        ],
        mesh=_sc_mesh(),
        compiler_params=pltpu.CompilerParams(needs_layout_passes=False),
        scratch_types=[
            pltpu.VMEM_SHARED((NP,), jnp.float32),
            pltpu.VMEM((NP,), jnp.float32),
            pltpu.VMEM((NP,), jnp.float32),
            pltpu.VMEM((KA, CH), jnp.int32),
            pltpu.VMEM((KA, CH), jnp.int32),
            pltpu.VMEM((KA, CH), jnp.float32),
            pltpu.VMEM((NPT,), jnp.float32),
        ],
    )(_weight_body)


def _weight_body(asv_hbm, adv_hbm, edges_hbm, w_hbm, den_hbm,
                 den_sh, as_l, ad_l, src_k, dst_k, w_k, zbuf1):
    cid = lax.axis_index("c")
    sid = lax.axis_index("s")
    rowbase = (cid * 16 + sid) * (CHT // 2)
    noa_l = CHT // 2 // KA

    pltpu.sync_copy(asv_hbm, as_l)
    pltpu.sync_copy(adv_hbm, ad_l)

    @pl.loop(0, NPT // 16)
    def _z1(i):
        zbuf1[pl.ds(i * 16, 16)] = jnp.zeros((16,), jnp.float32)

    nslice = pl.ds(sid * NPT, NPT)
    pltpu.sync_copy(zbuf1, den_sh.at[nslice])
    plsc.subcore_barrier()

    @pl.loop(0, noa_l)
    def _batch(o):
        bs = pl.ds((rowbase + o * KA), KA)
        pltpu.sync_copy(edges_hbm.at[0].at[bs], src_k)
        pltpu.sync_copy(edges_hbm.at[1].at[bs], dst_k)
        for jj in range(KA):
            base = (rowbase + o * KA + jj) * CH
            for g in range(CH // 16):
                s_idx = src_k[jj, pl.ds(g * 16, 16)]
                d_idx = dst_k[jj, pl.ds(g * 16, 16)]
                e = (plsc.load_gather(as_l, [s_idx])
                     + plsc.load_gather(ad_l, [d_idx]))
                e = jnp.maximum(e, 0.2 * e)
                w = jnp.exp(e)
                gid = base + g * 16 + lax.iota(jnp.int32, 16)
                w_k[jj, pl.ds(g * 16, 16)] = jnp.where(gid < E, w, 0.0)
        pltpu.sync_copy(w_k, w_hbm.at[bs])
        for jj in range(KA):
            pltpu.sync_copy(w_k.at[jj], den_sh.at[dst_k.at[jj]], add=True)

    plsc.subcore_barrier()
    pltpu.sync_copy(den_sh.at[nslice], zbuf1)
    pltpu.sync_copy(zbuf1, den_hbm.at[cid].at[nslice])


@functools.cache
def _make_accum_kernel():
    return functools.partial(
        pl.kernel,
        out_type=jax.ShapeDtypeStruct((2, NP, D), jnp.float32),
        mesh=_sc_mesh(),
        compiler_params=pltpu.CompilerParams(needs_layout_passes=False),
        scratch_types=[
            pltpu.VMEM_SHARED((NP, D), jnp.float32),
            pltpu.VMEM((KB, CH), jnp.int32),
            pltpu.VMEM((KB, CH), jnp.int32),
            pltpu.VMEM((KB, CH), jnp.int32),
            pltpu.VMEM((KB, CH), jnp.int32),
            pltpu.VMEM((KB, CH), jnp.float32),
            pltpu.VMEM((KB, CH), jnp.float32),
            pltpu.VMEM((CH, D), jnp.float32),
            pltpu.VMEM((CH, D), jnp.float32),
            pltpu.SemaphoreType.DMA,
            pltpu.SemaphoreType.DMA,
            pltpu.SemaphoreType.DMA,
            pltpu.SemaphoreType.DMA,
        ],
    )(_accum_body)


def _accum_body(h_hbm, w_hbm, edges_hbm, raw_hbm,
                acc, src0, src1, dst0, dst1, w0, w1, rows0, rows1,
                sg0, sg1, ss0, ss1):
    cid = lax.axis_index("c")
    sid = lax.axis_index("s")
    rowbase = sid * CHT + cid * CF
    nch_l = jnp.where(cid == 0, CF, CS)
    nob_l = nch_l // KB
    h_sel = h_hbm.at[cid]
    srcs, dsts, ws = (src0, src1), (dst0, dst1), (w0, w1)
    rows, sg, ss = (rows0, rows1), (sg0, sg1), (ss0, ss1)

    @pl.loop(0, CH)
    def _z(i):
        z = jnp.zeros((16,), jnp.float32)
        for g in range(D // 16):
            rows0[i, pl.ds(g * 16, 16)] = z

    for p in range(NPT // CH):
        pltpu.sync_copy(rows0, acc.at[pl.ds(sid * NPT + p * CH, CH)])
    plsc.subcore_barrier()

    def load_batch(b, s):
        bs = pl.ds((rowbase + b * KB), KB)
        pltpu.sync_copy(edges_hbm.at[0].at[bs], srcs[s])
        pltpu.sync_copy(edges_hbm.at[1].at[bs], dsts[s])
        pltpu.sync_copy(w_hbm.at[bs], ws[s])

    load_batch(0, 0)
    load_batch(1, 1)
    pltpu.async_copy(h_sel.at[srcs[0].at[0]], rows[0], sg[0])

    @pl.loop(0, nob_l // 2)
    def _pair(k):
        for half in range(2):
            b = 2 * k + half
            for jj in range(KB):
                j = b * KB + jj
                g = jj % 2
                og = 1 - g
                @pl.when(j > 0)
                def _w0():
                    pltpu.make_async_copy(
                        rows[og], acc.at[dsts[half].at[jj]], ss[og]).wait()
                if jj + 1 < KB:
                    nset, nrow = half, jj + 1
                else:
                    nset, nrow = 1 - half, 0

                @pl.when(j + 1 < nch_l)
                def _g1():
                    pltpu.async_copy(
                        h_sel.at[srcs[nset].at[nrow]], rows[og], sg[og])
                pltpu.make_async_copy(
                    h_sel.at[srcs[half].at[jj]], rows[g], sg[g]).wait()

                @pl.loop(0, CH // 16)
                def _scale(bb):
                    w16 = ws[half][jj, pl.ds(bb * 16, 16)]
                    for i in range(16):
                        s = w16[i]
                        r = bb * 16 + i
                        for gg in range(D // 16):
                            rows[g][r, pl.ds(gg * 16, 16)] = (
                                rows[g][r, pl.ds(gg * 16, 16)] * s)

                pltpu.async_copy(rows[g], acc.at[dsts[half].at[jj]],
                                 ss[g], add=True)
            @pl.when(b + 2 < nob_l)
            def _r2():
                load_batch(b + 2, half)

    pltpu.make_async_copy(rows[1], acc.at[dsts[1].at[KB - 1]], ss[1]).wait()
    plsc.subcore_barrier()

    for p in range(NPT // CH):
        rs = pl.ds(sid * NPT + p * CH, CH)
        pltpu.sync_copy(acc.at[rs], rows0)
        pltpu.sync_copy(rows0, raw_hbm.at[cid].at[rs])


def kernel(x, edge_index, W1, a_src1, a_dst1, b1, W2, a_src2, a_dst2, b2):
    ei = jnp.pad(edge_index.astype(jnp.int32), ((0, 0), (0, EP - E)))
    ei = ei.reshape(2, 16 * CHT, CH)
    xp = jnp.pad(x, ((0, NP - N), (0, 0)))

    weight_kernel = _make_weight_kernel()
    accum_kernel = _make_accum_kernel()

    h, asv, adv = _dense1(xp, W1, a_src1.reshape(D, 1), a_dst1.reshape(D, 1))
    wv, den = weight_kernel(asv.reshape(NP), adv.reshape(NP), ei)
    raw = accum_kernel(h, wv, ei)

    h2, asv2, adv2 = _dense2(raw, den.reshape(2, NP, 1), b1.reshape(1, D), W2,
                             a_src2.reshape(D, 1), a_dst2.reshape(D, 1))
    wv2, den2 = weight_kernel(asv2.reshape(NP), adv2.reshape(NP), ei)
    raw2 = accum_kernel(h2, wv2, ei)

    return _finalize(raw2[:, :N], den2.reshape(2, NP, 1)[:, :N],
                     b2.reshape(1, D))


# speedup vs baseline: 1.1344x; 1.0168x over previous
import functools

import jax
import jax.numpy as jnp
from jax import lax
from jax.experimental import pallas as pl
from jax.experimental.pallas import tpu as pltpu
from jax.experimental.pallas import tpu_sc as plsc

N = 10000
E = 320000
D = 128
NP = 10240
NPT = NP // 16
NW = 32
CH = 128
CHT = 160
CF = 120
CS = CHT - CF
EP = CHT * 16 * CH
KA = 8
KB = 4
BM = 1024
BN = 1000
EPS = 1e-16


def _prep_body(w1_ref, av1_ref, ad1_ref, w2_ref, av2_ref, ad2_ref, out_ref):
    out_ref[0] = jnp.dot(w1_ref[...], av1_ref[...],
                         preferred_element_type=jnp.float32)
    out_ref[1] = jnp.dot(w1_ref[...], ad1_ref[...],
                         preferred_element_type=jnp.float32)
    out_ref[2] = jnp.dot(w2_ref[...], av2_ref[...],
                         preferred_element_type=jnp.float32)
    out_ref[3] = jnp.dot(w2_ref[...], ad2_ref[...],
                         preferred_element_type=jnp.float32)


def _logits_body(x_ref, wv_ref, as_ref, ad_ref):
    as_ref[...] = jnp.dot(x_ref[...], wv_ref[0],
                          preferred_element_type=jnp.float32)
    ad_ref[...] = jnp.dot(x_ref[...], wv_ref[1],
                          preferred_element_type=jnp.float32)


def _matmul_body(x_ref, w_ref, h_ref):
    h = jnp.dot(x_ref[...], w_ref[...], preferred_element_type=jnp.float32)
    h_ref[0] = h
    h_ref[1] = h


def _norm_body(raw_ref, den_ref, b_ref, x_ref):
    raw = raw_ref[0] + raw_ref[1]
    den = den_ref[0] + den_ref[1]
    x_ref[...] = jnp.maximum(raw / (den + EPS) + b_ref[...], 0.0)


def _final_body(raw_ref, den_ref, b_ref, out_ref):
    raw = raw_ref[0] + raw_ref[1]
    den = den_ref[0] + den_ref[1]
    out_ref[...] = raw / (den + EPS) + b_ref[...]


def _prep(W1, av1, ad1, W2, av2, ad2):
    spec_w = pl.BlockSpec((D, D), lambda: (0, 0))
    spec_v = pl.BlockSpec((D, 1), lambda: (0, 0))
    return pl.pallas_call(
        _prep_body,
        grid=(),
        in_specs=[spec_w, spec_v, spec_v, spec_w, spec_v, spec_v],
        out_specs=pl.BlockSpec((4, D, 1), lambda: (0, 0, 0)),
        out_shape=jax.ShapeDtypeStruct((4, D, 1), jnp.float32),
    )(W1, av1, ad1, W2, av2, ad2)


def _logits(x, wv):
    return pl.pallas_call(
        _logits_body,
        grid=(NP // BM,),
        in_specs=[
            pl.BlockSpec((BM, D), lambda i: (i, 0)),
            pl.BlockSpec((2, D, 1), lambda i: (0, 0, 0)),
        ],
        out_specs=[
            pl.BlockSpec((BM, 1), lambda i: (i, 0)),
            pl.BlockSpec((BM, 1), lambda i: (i, 0)),
        ],
        out_shape=[
            jax.ShapeDtypeStruct((NP, 1), jnp.float32),
            jax.ShapeDtypeStruct((NP, 1), jnp.float32),
        ],
    )(x, wv)


def _matmul(x, W):
    return pl.pallas_call(
        _matmul_body,
        grid=(NP // BM,),
        in_specs=[
            pl.BlockSpec((BM, D), lambda i: (i, 0)),
            pl.BlockSpec((D, D), lambda i: (0, 0)),
        ],
        out_specs=pl.BlockSpec((2, BM, D), lambda i: (0, i, 0)),
        out_shape=jax.ShapeDtypeStruct((2, NP, D), jnp.float32),
    )(x, W)


def _norm(raw, den, b):
    return pl.pallas_call(
        _norm_body,
        grid=(NP // BM,),
        in_specs=[
            pl.BlockSpec((2, BM, D), lambda i: (0, i, 0)),
            pl.BlockSpec((2, BM, 1), lambda i: (0, i, 0)),
            pl.BlockSpec((1, D), lambda i: (0, 0)),
        ],
        out_specs=pl.BlockSpec((BM, D), lambda i: (i, 0)),
        out_shape=jax.ShapeDtypeStruct((NP, D), jnp.float32),
    )(raw, den, b)


def _finalize(raw, den, b):
    return pl.pallas_call(
        _final_body,
        grid=(N // BN,),
        in_specs=[
            pl.BlockSpec((2, BN, D), lambda i: (0, i, 0)),
            pl.BlockSpec((2, BN, 1), lambda i: (0, i, 0)),
            pl.BlockSpec((1, D), lambda i: (0, 0)),
        ],
        out_specs=pl.BlockSpec((BN, D), lambda i: (i, 0)),
        out_shape=jax.ShapeDtypeStruct((N, D), jnp.float32),
    )(raw, den, b)


@functools.cache
def _sc_mesh():
    return plsc.VectorSubcoreMesh(core_axis_name="c", subcore_axis_name="s",
                                  num_cores=2, num_subcores=16)


@functools.cache
def _make_weight_kernel():
    return functools.partial(
        pl.kernel,
        out_type=[
            jax.ShapeDtypeStruct((16 * CHT, CH), jnp.float32),
            jax.ShapeDtypeStruct((2, NP), jnp.float32),
        ],
        mesh=_sc_mesh(),
        compiler_params=pltpu.CompilerParams(needs_layout_passes=False),
        scratch_types=[
            pltpu.VMEM_SHARED((NP,), jnp.float32),
            pltpu.VMEM((NP,), jnp.float32),
            pltpu.VMEM((NP,), jnp.float32),
            pltpu.VMEM((KA, CH), jnp.int32),
            pltpu.VMEM((KA, CH), jnp.int32),
            pltpu.VMEM((KA, CH), jnp.float32),
            pltpu.VMEM((NPT,), jnp.float32),
        ],
    )(_weight_body)


def _weight_body(asv_hbm, adv_hbm, edges_hbm, w_hbm, den_hbm,
                 den_sh, as_l, ad_l, src_k, dst_k, w_k, zbuf1):
    cid = lax.axis_index("c")
    sid = lax.axis_index("s")
    rowbase = (cid * 16 + sid) * (CHT // 2)
    noa_l = CHT // 2 // KA

    pltpu.sync_copy(asv_hbm, as_l)
    pltpu.sync_copy(adv_hbm, ad_l)

    @pl.loop(0, NPT // 16)
    def _z1(i):
        zbuf1[pl.ds(i * 16, 16)] = jnp.zeros((16,), jnp.float32)

    nslice = pl.ds(sid * NPT, NPT)
    pltpu.sync_copy(zbuf1, den_sh.at[nslice])
    plsc.subcore_barrier()

    @pl.loop(0, noa_l)
    def _batch(o):
        bs = pl.ds((rowbase + o * KA), KA)
        pltpu.sync_copy(edges_hbm.at[0].at[bs], src_k)
        pltpu.sync_copy(edges_hbm.at[1].at[bs], dst_k)
        for jj in range(KA):
            base = (rowbase + o * KA + jj) * CH
            for g in range(CH // 16):
                s_idx = src_k[jj, pl.ds(g * 16, 16)]
                d_idx = dst_k[jj, pl.ds(g * 16, 16)]
                e = (plsc.load_gather(as_l, [s_idx])
                     + plsc.load_gather(ad_l, [d_idx]))
                e = jnp.maximum(e, 0.2 * e)
                w = jnp.exp(e)
                gid = base + g * 16 + lax.iota(jnp.int32, 16)
                w_k[jj, pl.ds(g * 16, 16)] = jnp.where(gid < E, w, 0.0)
        pltpu.sync_copy(w_k, w_hbm.at[bs])
        for jj in range(KA):
            pltpu.sync_copy(w_k.at[jj], den_sh.at[dst_k.at[jj]], add=True)

    plsc.subcore_barrier()
    pltpu.sync_copy(den_sh.at[nslice], zbuf1)
    pltpu.sync_copy(zbuf1, den_hbm.at[cid].at[nslice])


@functools.cache
def _make_accum_kernel():
    return functools.partial(
        pl.kernel,
        out_type=jax.ShapeDtypeStruct((2, NP, D), jnp.float32),
        mesh=_sc_mesh(),
        compiler_params=pltpu.CompilerParams(needs_layout_passes=False),
        scratch_types=[
            pltpu.VMEM_SHARED((NP, D), jnp.float32),
            pltpu.VMEM((KB, CH), jnp.int32),
            pltpu.VMEM((KB, CH), jnp.int32),
            pltpu.VMEM((KB, CH), jnp.int32),
            pltpu.VMEM((KB, CH), jnp.int32),
            pltpu.VMEM((KB, CH), jnp.float32),
            pltpu.VMEM((KB, CH), jnp.float32),
            pltpu.VMEM((CH, D), jnp.float32),
            pltpu.VMEM((CH, D), jnp.float32),
            pltpu.SemaphoreType.DMA,
            pltpu.SemaphoreType.DMA,
            pltpu.SemaphoreType.DMA,
            pltpu.SemaphoreType.DMA,
        ],
    )(_accum_body)


def _accum_body(h_hbm, w_hbm, edges_hbm, raw_hbm,
                acc, src0, src1, dst0, dst1, w0, w1, rows0, rows1,
                sg0, sg1, ss0, ss1):
    cid = lax.axis_index("c")
    sid = lax.axis_index("s")
    rowbase = sid * CHT + cid * CF
    nch_l = jnp.where(cid == 0, CF, CS)
    nob_l = nch_l // KB
    h_sel = h_hbm.at[cid]
    srcs, dsts, ws = (src0, src1), (dst0, dst1), (w0, w1)
    rows, sg, ss = (rows0, rows1), (sg0, sg1), (ss0, ss1)

    @pl.loop(0, CH)
    def _z(i):
        z = jnp.zeros((16,), jnp.float32)
        for g in range(D // 16):
            rows0[i, pl.ds(g * 16, 16)] = z

    for p in range(NPT // CH):
        pltpu.sync_copy(rows0, acc.at[pl.ds(sid * NPT + p * CH, CH)])
    plsc.subcore_barrier()

    def load_batch(b, s):
        bs = pl.ds((rowbase + b * KB), KB)
        pltpu.sync_copy(edges_hbm.at[0].at[bs], srcs[s])
        pltpu.sync_copy(edges_hbm.at[1].at[bs], dsts[s])
        pltpu.sync_copy(w_hbm.at[bs], ws[s])

    load_batch(0, 0)
    load_batch(1, 1)
    pltpu.async_copy(h_sel.at[srcs[0].at[0]], rows[0], sg[0])

    @pl.loop(0, nob_l // 2)
    def _pair(k):
        for half in range(2):
            b = 2 * k + half
            for jj in range(KB):
                j = b * KB + jj
                g = jj % 2
                og = 1 - g
                @pl.when(j > 0)
                def _w0():
                    pltpu.make_async_copy(
                        rows[og], acc.at[dsts[half].at[jj]], ss[og]).wait()
                if jj + 1 < KB:
                    nset, nrow = half, jj + 1
                else:
                    nset, nrow = 1 - half, 0

                @pl.when(j + 1 < nch_l)
                def _g1():
                    pltpu.async_copy(
                        h_sel.at[srcs[nset].at[nrow]], rows[og], sg[og])
                pltpu.make_async_copy(
                    h_sel.at[srcs[half].at[jj]], rows[g], sg[g]).wait()

                @pl.loop(0, CH // 16)
                def _scale(bb):
                    w16 = ws[half][jj, pl.ds(bb * 16, 16)]
                    for i in range(16):
                        s = w16[i]
                        r = bb * 16 + i
                        for gg in range(D // 16):
                            rows[g][r, pl.ds(gg * 16, 16)] = (
                                rows[g][r, pl.ds(gg * 16, 16)] * s)

                pltpu.async_copy(rows[g], acc.at[dsts[half].at[jj]],
                                 ss[g], add=True)
            @pl.when(b + 2 < nob_l)
            def _r2():
                load_batch(b + 2, half)

    pltpu.make_async_copy(rows[1], acc.at[dsts[1].at[KB - 1]], ss[1]).wait()
    plsc.subcore_barrier()

    for p in range(NPT // CH):
        rs = pl.ds(sid * NPT + p * CH, CH)
        pltpu.sync_copy(acc.at[rs], rows0)
        pltpu.sync_copy(rows0, raw_hbm.at[cid].at[rs])


def kernel(x, edge_index, W1, a_src1, a_dst1, b1, W2, a_src2, a_dst2, b2):
    ei = jnp.pad(edge_index.astype(jnp.int32), ((0, 0), (0, EP - E)))
    ei = ei.reshape(2, 16 * CHT, CH)
    xp = jnp.pad(x, ((0, NP - N), (0, 0)))

    weight_kernel = _make_weight_kernel()
    accum_kernel = _make_accum_kernel()

    wv = _prep(W1, a_src1.reshape(D, 1), a_dst1.reshape(D, 1),
               W2, a_src2.reshape(D, 1), a_dst2.reshape(D, 1))

    asv, adv = _logits(xp, wv[:2])
    wgt, den = weight_kernel(asv.reshape(NP), adv.reshape(NP), ei)
    h = _matmul(xp, W1)
    raw = accum_kernel(h, wgt, ei)

    x2 = _norm(raw, den.reshape(2, NP, 1), b1.reshape(1, D))
    asv2, adv2 = _logits(x2, wv[2:])
    wgt2, den2 = weight_kernel(asv2.reshape(NP), adv2.reshape(NP), ei)
    h2 = _matmul(x2, W2)
    raw2 = accum_kernel(h2, wgt2, ei)

    return _finalize(raw2[:, :N], den2.reshape(2, NP, 1)[:, :N],
                     b2.reshape(1, D))
